# baseline (speedup 1.0000x reference)
import functools

import jax
import jax.numpy as jnp
from jax import lax
from jax.experimental import pallas as pl
from jax.experimental.pallas import tpu as pltpu
from jax.experimental.pallas import tpu_sc as plsc

NBASIS = 1000000
NUNIQ = 65536

_NC = 2
_NS = 16
_NW = _NC * _NS
_LANES = 16

_PER_TILE = 31296
_NCH = 4
_CHUNK = _PER_TILE // _NCH
_NP = _NW * _PER_TILE

_mesh = plsc.VectorSubcoreMesh(core_axis_name="c", subcore_axis_name="s")


@functools.partial(
    pl.kernel,
    mesh=_mesh,
    out_type=jax.ShapeDtypeStruct((_NP,), jnp.float32),
    scratch_types=[
        pltpu.VMEM((NUNIQ,), jnp.float32),
        pltpu.VMEM((_PER_TILE,), jnp.int32),
        pltpu.VMEM((_PER_TILE,), jnp.float32),
        pltpu.VMEM((_LANES,), jnp.float32),
        pltpu.SemaphoreType.DMA,
        pltpu.SemaphoreType.DMA,
        pltpu.SemaphoreType.DMA,
    ],
    compiler_params=pltpu.CompilerParams(needs_layout_passes=False),
)
def _sc_gather(ke_hbm, idx_hbm, step_hbm, out_hbm,
               tab_v, idx_v, out_v, step_v,
               sem_tab, sem_in, sem_out):
    c = lax.axis_index("c")
    s = lax.axis_index("s")
    wid = s * _NC + c
    base = wid * _PER_TILE

    tab_copy = pltpu.async_copy(ke_hbm, tab_v, sem_tab)
    in_copies = [
        pltpu.async_copy(
            idx_hbm.at[pl.ds(base + k * _CHUNK, _CHUNK)],
            idx_v.at[pl.ds(k * _CHUNK, _CHUNK)],
            sem_in,
        )
        for k in range(_NCH)
    ]
    pltpu.sync_copy(step_hbm, step_v)
    tab_copy.wait()
    sv = step_v[...]

    out_copies = []
    for k in range(_NCH):
        in_copies[k].wait()
        kbase = k * _CHUNK

        def body(i, _, kbase=kbase):
            off = pl.multiple_of(kbase + i * _LANES, _LANES)
            iv = idx_v[pl.ds(off, _LANES)]
            vals = plsc.load_gather(tab_v, [iv])
            out_v[pl.ds(off, _LANES)] = vals * sv
            return 0

        lax.fori_loop(0, _CHUNK // _LANES, body, 0)
        out_copies.append(
            pltpu.async_copy(
                out_v.at[pl.ds(kbase, _CHUNK)],
                out_hbm.at[pl.ds(base + kbase, _CHUNK)],
                sem_out,
            )
        )
    for cp in out_copies:
        cp.wait()


def kernel(ke, ke_invidx, step):
    idx = jnp.pad(ke_invidx.astype(jnp.int32), (0, _NP - NBASIS))
    step_vec = jnp.full((_LANES,), step, dtype=jnp.float32)
    out = _sc_gather(ke, idx, step_vec)
    return out[:NBASIS]


# speedup vs baseline: 1.1715x; 1.1715x over previous
import functools

import jax
import jax.numpy as jnp
from jax import lax
from jax.experimental import pallas as pl
from jax.experimental.pallas import tpu as pltpu
from jax.experimental.pallas import tpu_sc as plsc

NBASIS = 1000000
NUNIQ = 65536

_NC = 2
_NS = 16
_NW = _NC * _NS
_LANES = 16

_FULL = 31264
_TAIL = NBASIS - (_NW - 1) * _FULL

_mesh = plsc.VectorSubcoreMesh(core_axis_name="c", subcore_axis_name="s")


@functools.partial(
    pl.kernel,
    mesh=_mesh,
    out_type=jax.ShapeDtypeStruct((NBASIS,), jnp.float32),
    scratch_types=[
        pltpu.VMEM((NUNIQ,), jnp.float32),
        pltpu.VMEM((_FULL,), jnp.int32),
        pltpu.VMEM((_FULL,), jnp.float32),
        pltpu.VMEM((_LANES,), jnp.float32),
        pltpu.SemaphoreType.DMA,
        pltpu.SemaphoreType.DMA,
    ],
    compiler_params=pltpu.CompilerParams(needs_layout_passes=False),
)
def _sc_gather(ke_hbm, idx_hbm, step_hbm, out_hbm,
               tab_v, idx_v, out_v, step_v, sem_tab, sem_in):
    c = lax.axis_index("c")
    s = lax.axis_index("s")
    wid = s * _NC + c
    base = wid * _FULL

    tab_cp = pltpu.async_copy(ke_hbm, tab_v, sem_tab)
    pltpu.sync_copy(step_hbm, step_v)

    def work(n):
        in_cp = pltpu.async_copy(
            idx_hbm.at[pl.ds(base, n)], idx_v.at[pl.ds(0, n)], sem_in
        )
        tab_cp.wait()
        in_cp.wait()
        sv = step_v[...]

        def body(i, _):
            off = pl.multiple_of(i * _LANES, _LANES)
            iv = idx_v[pl.ds(off, _LANES)]
            out_v[pl.ds(off, _LANES)] = plsc.load_gather(tab_v, [iv]) * sv
            return 0

        lax.fori_loop(0, n // _LANES, body, 0)
        pltpu.sync_copy(out_v.at[pl.ds(0, n)], out_hbm.at[pl.ds(base, n)])

    @pl.when(wid != _NW - 1)
    def _():
        work(_FULL)

    @pl.when(wid == _NW - 1)
    def _():
        work(_TAIL)


def kernel(ke, ke_invidx, step):
    idx = ke_invidx.astype(jnp.int32)
    step_vec = jnp.full((_LANES,), step, dtype=jnp.float32)
    return _sc_gather(ke, idx, step_vec)


# speedup vs baseline: 1.4757x; 1.2597x over previous
import functools

import jax
import jax.numpy as jnp
from jax import lax
from jax.experimental import pallas as pl
from jax.experimental.pallas import tpu as pltpu
from jax.experimental.pallas import tpu_sc as plsc

NBASIS = 1000000
NUNIQ = 65536

_NC = 2
_NS = 16
_NW = _NC * _NS
_LANES = 16

_FULL = 31264
_TAIL = NBASIS - (_NW - 1) * _FULL

_mesh = plsc.VectorSubcoreMesh(core_axis_name="c", subcore_axis_name="s")


@functools.partial(
    pl.kernel,
    mesh=_mesh,
    out_type=jax.ShapeDtypeStruct((NBASIS,), jnp.float32),
    scratch_types=[
        pltpu.VMEM((NUNIQ,), jnp.float32),
        pltpu.VMEM((_FULL,), jnp.int32),
        pltpu.VMEM((_FULL,), jnp.float32),
        pltpu.VMEM((_LANES,), jnp.float32),
        pltpu.SemaphoreType.DMA,
        pltpu.SemaphoreType.DMA,
    ],
    compiler_params=pltpu.CompilerParams(needs_layout_passes=False),
)
def _sc_gather(ke_hbm, idx_hbm, step_hbm, out_hbm,
               tab_v, idx_v, out_v, step_v, sem_tab, sem_in):
    c = lax.axis_index("c")
    s = lax.axis_index("s")
    wid = s * _NC + c
    base = wid * _FULL

    tab_cp = pltpu.async_copy(ke_hbm, tab_v, sem_tab)
    pltpu.sync_copy(step_hbm, step_v)

    def work(n):
        in_cp = pltpu.async_copy(
            idx_hbm.at[pl.ds(base, n)], idx_v.at[pl.ds(0, n)], sem_in
        )
        tab_cp.wait()
        in_cp.wait()
        sv = step_v[...]

        @plsc.parallel_loop(0, n // _LANES, unroll=8)
        def body(i):
            off = pl.multiple_of(i * _LANES, _LANES)
            iv = idx_v[pl.ds(off, _LANES)]
            out_v[pl.ds(off, _LANES)] = plsc.load_gather(tab_v, [iv]) * sv
        pltpu.sync_copy(out_v.at[pl.ds(0, n)], out_hbm.at[pl.ds(base, n)])

    @pl.when(wid != _NW - 1)
    def _():
        work(_FULL)

    @pl.when(wid == _NW - 1)
    def _():
        work(_TAIL)


def kernel(ke, ke_invidx, step):
    idx = ke_invidx.astype(jnp.int32)
    step_vec = jnp.full((_LANES,), step, dtype=jnp.float32)
    return _sc_gather(ke, idx, step_vec)
